# baseline (speedup 1.0000x reference)
import functools

import jax
import jax.numpy as jnp
from jax import lax
from jax.experimental import pallas as pl
from jax.experimental.pallas import tpu as pltpu
from jax.experimental.pallas import tpu_sc as plsc

N = 10000
E = 320000
NFEAT = 128
NP = 10112
CHUNK = 128
NCHUNKS = 2560
E_PAD = NCHUNKS * CHUNK
NLANE = 16
RPT = NP // 16
ZR = 158

f32 = jnp.float32
i32 = jnp.int32


def _mesh():
    return plsc.VectorSubcoreMesh(core_axis_name="c", subcore_axis_name="s")


def _make_deg():
    K = NCHUNKS // 32

    @functools.partial(
        pl.kernel,
        out_type=(jax.ShapeDtypeStruct((32, NP), f32),
                  jax.ShapeDtypeStruct((32, NP), f32)),
        mesh=_mesh(),
        scratch_types=[
            pltpu.VMEM((K * CHUNK,), i32),
            pltpu.VMEM((K * CHUNK,), i32),
            pltpu.VMEM((8 * NP,), f32),
            pltpu.VMEM((NP,), f32),
        ],
        compiler_params=pltpu.CompilerParams(needs_layout_passes=False),
    )
    def deg(src_hbm, dst_hbm, hs_out, hd_out, idx_s, idx_d, hist, osum):
        c = lax.axis_index("c")
        s = lax.axis_index("s")
        wid = s * 2 + c
        base = wid * (K * CHUNK)
        pltpu.sync_copy(src_hbm.at[pl.ds(base, K * CHUNK)], idx_s)
        pltpu.sync_copy(dst_hbm.at[pl.ds(base, K * CHUNK)], idx_d)

        def zstep(i, carry):
            hist[pl.ds(i * NLANE, NLANE)] = jnp.zeros((NLANE,), f32)
            return carry

        lax.fori_loop(0, (8 * NP) // NLANE, zstep, 0)

        iota = lax.iota(i32, NLANE)
        rowoff = (iota % 4) * NP
        ones = jnp.ones((NLANE,), f32)
        masks = [(iota >= q * 4) & (iota < q * 4 + 4) for q in range(4)]

        def estep(j, carry):
            off = j * CHUNK
            for g in range(CHUNK // NLANE):
                sv = idx_s[pl.ds(off + g * NLANE, NLANE)]
                dv = idx_d[pl.ds(off + g * NLANE, NLANE)]
                fs = rowoff + sv
                fd = rowoff + dv + 4 * NP
                for q in range(4):
                    plsc.addupdate_scatter(hist, [fs], ones, mask=masks[q])
                    plsc.addupdate_scatter(hist, [fd], ones, mask=masks[q])
            return carry

        lax.fori_loop(0, K, estep, 0)

        def rstep_src(i, carry):
            off = i * NLANE
            a = (hist[pl.ds(off, NLANE)] + hist[pl.ds(NP + off, NLANE)]
                 + hist[pl.ds(2 * NP + off, NLANE)]
                 + hist[pl.ds(3 * NP + off, NLANE)])
            osum[pl.ds(off, NLANE)] = a
            return carry

        lax.fori_loop(0, NP // NLANE, rstep_src, 0)
        pltpu.sync_copy(osum, hs_out.at[wid])

        def rstep_dst(i, carry):
            off = i * NLANE
            a = (hist[pl.ds(4 * NP + off, NLANE)]
                 + hist[pl.ds(5 * NP + off, NLANE)]
                 + hist[pl.ds(6 * NP + off, NLANE)]
                 + hist[pl.ds(7 * NP + off, NLANE)])
            osum[pl.ds(off, NLANE)] = a
            return carry

        lax.fori_loop(0, NP // NLANE, rstep_dst, 0)
        pltpu.sync_copy(osum, hd_out.at[wid])

    return deg


IB = 8


def _make_agg(D, col_split):
    K = NCHUNKS // 16 if col_split else NCHUNKS // 32

    @functools.partial(
        pl.kernel,
        out_type=jax.ShapeDtypeStruct((2, NP, D), f32),
        mesh=_mesh(),
        scratch_types=[
            pltpu.VMEM((IB, CHUNK), i32),
            pltpu.VMEM((IB, CHUNK), i32),
            pltpu.VMEM((CHUNK, D), f32),
            pltpu.VMEM((CHUNK, D), f32),
            pltpu.VMEM_SHARED((NP, D), f32),
            pltpu.SemaphoreType.DMA,
            pltpu.SemaphoreType.DMA,
            pltpu.SemaphoreType.DMA,
            pltpu.SemaphoreType.DMA,
        ],
    )
    def agg(tbl0, tbl1, src_hbm, dst_hbm, zeros_hbm, out,
            idx_s, idx_d, rows0, rows1, aggm, gsem0, gsem1, ssem0, ssem1):
        c = lax.axis_index("c")
        s = lax.axis_index("s")
        if col_split:
            base = s * K
        else:
            base = (s * 2 + c) * K
        for q in range(RPT // ZR):
            pltpu.sync_copy(zeros_hbm, aggm.at[pl.ds(s * RPT + q * ZR, ZR)])
        plsc.subcore_barrier()

        rows = (rows0, rows1)
        gsem = (gsem0, gsem1)
        ssem = (ssem0, ssem1)

        def run(tbl):
            def outer(t, carry):
                pltpu.sync_copy(src_hbm.at[pl.ds(base + t * IB, IB)], idx_s)
                pltpu.sync_copy(dst_hbm.at[pl.ds(base + t * IB, IB)], idx_d)
                g = [None] * IB
                sd = [None] * IB
                g[0] = pltpu.async_copy(tbl.at[idx_s.at[0]], rows[0], gsem[0])
                for j in range(IB):
                    if j >= 1:
                        sd[j - 1].wait()
                    if j + 1 < IB:
                        g[j + 1] = pltpu.async_copy(
                            tbl.at[idx_s.at[j + 1]], rows[(j + 1) % 2],
                            gsem[(j + 1) % 2])
                    g[j].wait()
                    sd[j] = pltpu.async_copy(
                        rows[j % 2], aggm.at[idx_d.at[j]], ssem[j % 2],
                        add=True)
                sd[IB - 1].wait()
                return carry
            lax.fori_loop(0, K // IB, outer, 0)

        @pl.when(c == 0)
        def _():
            run(tbl0)

        @pl.when(c == 1)
        def _():
            run(tbl1)

        plsc.subcore_barrier()
        pltpu.sync_copy(aggm.at[pl.ds(s * RPT, RPT)],
                        out.at[c, pl.ds(s * RPT, RPT)])

    return agg


_deg = _make_deg()
_agg128 = _make_agg(128, False)
_agg128c = _make_agg(128, True)
_agg64 = _make_agg(128, False)


def _p0_body(hs_ref, hd_ref, x_ref, dds_ref, ddd_ref, xs_ref, xs2_ref):
    rs = lax.rsqrt(jnp.maximum(jnp.sum(hs_ref[...], axis=0), 1.0))
    rd = lax.rsqrt(jnp.maximum(jnp.sum(hd_ref[...], axis=0), 1.0))
    dsb = jnp.transpose(jnp.broadcast_to(rs[None, :], (128, 128)))
    ddb = jnp.transpose(jnp.broadcast_to(rd[None, :], (128, 128)))
    dds_ref[...] = dsb[:, :8]
    ddd_ref[...] = ddb[:, :8]
    xs = x_ref[...] * dsb
    xs_ref[...] = xs
    xs2_ref[...] = xs


def _p0(hs, hd, x_p):
    return pl.pallas_call(
        _p0_body,
        grid=(NP // 128,),
        in_specs=[
            pl.BlockSpec((32, 128), lambda i: (0, i)),
            pl.BlockSpec((32, 128), lambda i: (0, i)),
            pl.BlockSpec((128, 128), lambda i: (i, 0)),
        ],
        out_specs=[
            pl.BlockSpec((128, 8), lambda i: (i, 0)),
            pl.BlockSpec((128, 8), lambda i: (i, 0)),
            pl.BlockSpec((128, 128), lambda i: (i, 0)),
            pl.BlockSpec((128, 128), lambda i: (i, 0)),
        ],
        out_shape=[jax.ShapeDtypeStruct((NP, 8), f32)] * 2
        + [jax.ShapeDtypeStruct((NP, 128), f32)] * 2,
    )(hs, hd, x_p)


def _t1_body(u0, u1, ddd, dds, w1, b1, w2, s2a, s2b):
    z = (u0[...] + u1[...]) * ddd[...][:, 0:1]
    h = jnp.maximum(
        jnp.dot(z, w1[...], preferred_element_type=f32) + b1[...], 0.0)
    s2 = jnp.dot(h, w2[...], preferred_element_type=f32) * dds[...][:, 0:1]
    s2a[...] = s2[:, :128]
    s2b[...] = s2[:, 128:]


def _t1(u0, u1, ddd, dds, w1, b1, w2):
    blk = lambda sh: pl.BlockSpec(sh, lambda i: (i, 0))
    full = lambda sh: pl.BlockSpec(sh, lambda i: (0, 0))
    return pl.pallas_call(
        _t1_body,
        grid=(NP // 128,),
        in_specs=[blk((128, 128)), blk((128, 128)), blk((128, 8)),
                  blk((128, 8)), full((128, 512)), full((1, 512)),
                  full((512, 256))],
        out_specs=[blk((128, 128)), blk((128, 128))],
        out_shape=[jax.ShapeDtypeStruct((NP, 128), f32)] * 2,
    )(u0, u1, ddd, dds, w1, b1, w2)


def _t2_body(u0, u1, ddd, dds, w3, b2, s3_ref, s3b_ref):
    dd = ddd[...][:, 0:1]
    ha = jnp.maximum(u0[...] * dd + b2[...][:, :128], 0.0)
    hb = jnp.maximum(u1[...] * dd + b2[...][:, 128:], 0.0)
    s3 = (jnp.dot(ha, w3[...][:128, :], preferred_element_type=f32)
          + jnp.dot(hb, w3[...][128:, :], preferred_element_type=f32))
    s3v = s3 * dds[...][:, 0:1]
    s3_ref[...] = s3v
    s3b_ref[...] = s3v


def _t2(u0, u1, ddd, dds, w3, b2):
    blk = lambda sh: pl.BlockSpec(sh, lambda i: (i, 0))
    full = lambda sh: pl.BlockSpec(sh, lambda i: (0, 0))
    return pl.pallas_call(
        _t2_body,
        grid=(NP // 128,),
        in_specs=[blk((128, 128)), blk((128, 128)), blk((128, 8)),
                  blk((128, 8)), full((256, 128)), full((1, 256))],
        out_specs=[blk((128, 128)), blk((128, 128))],
        out_shape=[jax.ShapeDtypeStruct((NP, 128), f32)] * 2,
    )(u0, u1, ddd, dds, w3, b2)


def _t3_body(u0, u1, ddd, dds, w4, b3, s4_ref, s4b_ref):
    h = jnp.maximum((u0[...] + u1[...]) * ddd[...][:, 0:1] + b3[...], 0.0)
    s4 = jnp.dot(h, w4[...], preferred_element_type=f32) * dds[...][:, 0:1]
    s4p = jnp.concatenate([s4, jnp.zeros((128, 64), f32)], axis=1)
    s4_ref[...] = s4p
    s4b_ref[...] = s4p


def _t3(u0, u1, ddd, dds, w4, b3):
    blk = lambda sh: pl.BlockSpec(sh, lambda i: (i, 0))
    full = lambda sh: pl.BlockSpec(sh, lambda i: (0, 0))
    return pl.pallas_call(
        _t3_body,
        grid=(NP // 128,),
        in_specs=[blk((128, 128)), blk((128, 128)), blk((128, 8)),
                  blk((128, 8)), full((128, 64)), full((1, 128))],
        out_specs=[blk((128, 128)), blk((128, 128))],
        out_shape=[jax.ShapeDtypeStruct((NP, 128), f32)] * 2,
    )(u0, u1, ddd, dds, w4, b3)


def _t4_body(u0, u1, ddd, b4, out_ref):
    u = u0[...][:, :64] + u1[...][:, :64]
    out_ref[...] = u * ddd[...][:, 0:1] + b4[...]


def _t4(u0, u1, ddd, b4):
    blk = lambda sh: pl.BlockSpec(sh, lambda i: (i, 0))
    full = lambda sh: pl.BlockSpec(sh, lambda i: (0, 0))
    return pl.pallas_call(
        _t4_body,
        grid=(NP // 128,),
        in_specs=[blk((128, 128)), blk((128, 128)), blk((128, 8)),
                  full((1, 64))],
        out_specs=blk((128, 64)),
        out_shape=jax.ShapeDtypeStruct((NP, 64), f32),
    )(u0, u1, ddd, b4)


def kernel(x, edge_index, W1, b1, W2, b2, W3, b3, W4, b4):
    src = edge_index[0].astype(i32)
    dst = edge_index[1].astype(i32)
    pad = N + (jnp.arange(E_PAD - E, dtype=i32) % (NP - N))
    src_f = jnp.concatenate([src, pad])
    dst_f = jnp.concatenate([dst, pad])
    src2 = src_f.reshape(NCHUNKS, CHUNK)
    dst2 = dst_f.reshape(NCHUNKS, CHUNK)
    x_p = jnp.zeros((NP, NFEAT), f32).at[:N, :].set(x)
    z128 = jnp.zeros((ZR, 128), f32)


    hs, hd = _deg(src_f, dst_f)
    dds, ddd, xs, xs2 = _p0(hs, hd, x_p)
    u1 = _agg128(xs, xs2, src2, dst2, z128)
    s2a, s2b = _t1(u1[0], u1[1], ddd, dds, W1, b1.reshape(1, -1), W2)
    u2 = _agg128c(s2a, s2b, src2, dst2, z128)
    s3, s3b = _t2(u2[0], u2[1], ddd, dds, W3, b2.reshape(1, -1))
    u3 = _agg128(s3, s3b, src2, dst2, z128)
    s4, s4b = _t3(u3[0], u3[1], ddd, dds, W4, b3.reshape(1, -1))
    u4 = _agg64(s4, s4b, src2, dst2, z128)
    emb = _t4(u4[0], u4[1], ddd, b4.reshape(1, -1))
    return emb[:N]


# speedup vs baseline: 1.1934x; 1.1934x over previous
import functools

import jax
import jax.numpy as jnp
from jax import lax
from jax.experimental import pallas as pl
from jax.experimental.pallas import tpu as pltpu
from jax.experimental.pallas import tpu_sc as plsc

N = 10000
E = 320000
NFEAT = 128
NP = 10112
CHUNK = 128
NCHUNKS = 2560
E_PAD = NCHUNKS * CHUNK
NLANE = 16
RPT = NP // 16
ZR = 158

f32 = jnp.float32
i32 = jnp.int32


def _mesh():
    return plsc.VectorSubcoreMesh(core_axis_name="c", subcore_axis_name="s")


def _make_deg():
    K = NCHUNKS // 32

    @functools.partial(
        pl.kernel,
        out_type=(jax.ShapeDtypeStruct((32, NP), f32),
                  jax.ShapeDtypeStruct((32, NP), f32)),
        mesh=_mesh(),
        scratch_types=[
            pltpu.VMEM((K * CHUNK,), i32),
            pltpu.VMEM((K * CHUNK,), i32),
            pltpu.VMEM((8 * NP,), f32),
            pltpu.VMEM((NP,), f32),
        ],
        compiler_params=pltpu.CompilerParams(needs_layout_passes=False),
    )
    def deg(src_hbm, dst_hbm, hs_out, hd_out, idx_s, idx_d, hist, osum):
        c = lax.axis_index("c")
        s = lax.axis_index("s")
        wid = s * 2 + c
        base = wid * (K * CHUNK)
        pltpu.sync_copy(src_hbm.at[pl.ds(base, K * CHUNK)], idx_s)
        pltpu.sync_copy(dst_hbm.at[pl.ds(base, K * CHUNK)], idx_d)

        def zstep(i, carry):
            hist[pl.ds(i * NLANE, NLANE)] = jnp.zeros((NLANE,), f32)
            return carry

        lax.fori_loop(0, (8 * NP) // NLANE, zstep, 0)

        iota = lax.iota(i32, NLANE)
        rowoff = (iota % 4) * NP
        ones = jnp.ones((NLANE,), f32)
        masks = [(iota >= q * 4) & (iota < q * 4 + 4) for q in range(4)]

        def estep(j, carry):
            off = j * CHUNK
            for g in range(CHUNK // NLANE):
                sv = idx_s[pl.ds(off + g * NLANE, NLANE)]
                dv = idx_d[pl.ds(off + g * NLANE, NLANE)]
                fs = rowoff + sv
                fd = rowoff + dv + 4 * NP
                for q in range(4):
                    plsc.addupdate_scatter(hist, [fs], ones, mask=masks[q])
                    plsc.addupdate_scatter(hist, [fd], ones, mask=masks[q])
            return carry

        lax.fori_loop(0, K, estep, 0)

        def rstep_src(i, carry):
            off = i * NLANE
            a = (hist[pl.ds(off, NLANE)] + hist[pl.ds(NP + off, NLANE)]
                 + hist[pl.ds(2 * NP + off, NLANE)]
                 + hist[pl.ds(3 * NP + off, NLANE)])
            osum[pl.ds(off, NLANE)] = a
            return carry

        lax.fori_loop(0, NP // NLANE, rstep_src, 0)
        pltpu.sync_copy(osum, hs_out.at[wid])

        def rstep_dst(i, carry):
            off = i * NLANE
            a = (hist[pl.ds(4 * NP + off, NLANE)]
                 + hist[pl.ds(5 * NP + off, NLANE)]
                 + hist[pl.ds(6 * NP + off, NLANE)]
                 + hist[pl.ds(7 * NP + off, NLANE)])
            osum[pl.ds(off, NLANE)] = a
            return carry

        lax.fori_loop(0, NP // NLANE, rstep_dst, 0)
        pltpu.sync_copy(osum, hd_out.at[wid])

    return deg


IB = 8


def _make_agg(D, col_split):
    K = NCHUNKS // 16 if col_split else NCHUNKS // 32

    @functools.partial(
        pl.kernel,
        out_type=jax.ShapeDtypeStruct((2, NP, D), f32),
        mesh=_mesh(),
        scratch_types=[
            pltpu.VMEM((IB, CHUNK), i32),
            pltpu.VMEM((IB, CHUNK), i32),
            pltpu.VMEM((CHUNK, D), f32),
            pltpu.VMEM((CHUNK, D), f32),
            pltpu.VMEM_SHARED((NP, D), f32),
            pltpu.SemaphoreType.DMA,
            pltpu.SemaphoreType.DMA,
            pltpu.SemaphoreType.DMA,
            pltpu.SemaphoreType.DMA,
        ],
    )
    def agg(tbl0, tbl1, src_hbm, dst_hbm, zeros_hbm, out,
            idx_s, idx_d, rows0, rows1, aggm, gsem0, gsem1, ssem0, ssem1):
        c = lax.axis_index("c")
        s = lax.axis_index("s")
        if col_split:
            base = s * K
        else:
            base = (s * 2 + c) * K
        for q in range(RPT // ZR):
            pltpu.sync_copy(zeros_hbm, aggm.at[pl.ds(s * RPT + q * ZR, ZR)])
        plsc.subcore_barrier()

        rows = (rows0, rows1)
        gsem = (gsem0, gsem1)
        ssem = (ssem0, ssem1)

        def run(tbl):
            def outer(t, carry):
                pltpu.sync_copy(src_hbm.at[pl.ds(base + t * IB, IB)], idx_s)
                pltpu.sync_copy(dst_hbm.at[pl.ds(base + t * IB, IB)], idx_d)
                g = [None] * IB
                sd = [None] * IB
                g[0] = pltpu.async_copy(tbl.at[idx_s.at[0]], rows[0], gsem[0])
                for j in range(IB):
                    if j >= 1:
                        sd[j - 1].wait()
                    if j + 1 < IB:
                        g[j + 1] = pltpu.async_copy(
                            tbl.at[idx_s.at[j + 1]], rows[(j + 1) % 2],
                            gsem[(j + 1) % 2])
                    g[j].wait()
                    sd[j] = pltpu.async_copy(
                        rows[j % 2], aggm.at[idx_d.at[j]], ssem[j % 2],
                        add=True)
                sd[IB - 1].wait()
                return carry
            lax.fori_loop(0, K // IB, outer, 0)

        @pl.when(c == 0)
        def _():
            run(tbl0)

        @pl.when(c == 1)
        def _():
            run(tbl1)

        plsc.subcore_barrier()
        pltpu.sync_copy(aggm.at[pl.ds(s * RPT, RPT)],
                        out.at[c, pl.ds(s * RPT, RPT)])

    return agg


_deg = _make_deg()
_agg128 = _make_agg(128, False)
_agg128c = _make_agg(128, True)
_agg64 = _make_agg(128, False)


def _p0_body(hs_ref, hd_ref, x_ref, dds_ref, ddd_ref, xs_ref, xs2_ref):
    rs = lax.rsqrt(jnp.maximum(jnp.sum(hs_ref[...], axis=0), 1.0))
    rd = lax.rsqrt(jnp.maximum(jnp.sum(hd_ref[...], axis=0), 1.0))
    dsb = jnp.transpose(jnp.broadcast_to(rs[None, :], (128, 128)))
    ddb = jnp.transpose(jnp.broadcast_to(rd[None, :], (128, 128)))
    dds_ref[...] = dsb[:, :8]
    ddd_ref[...] = ddb[:, :8]
    xs = x_ref[...] * dsb
    xs_ref[...] = xs
    xs2_ref[...] = xs


def _p0(hs, hd, x_p):
    return pl.pallas_call(
        _p0_body,
        grid=(NP // 128,),
        in_specs=[
            pl.BlockSpec((32, 128), lambda i: (0, i)),
            pl.BlockSpec((32, 128), lambda i: (0, i)),
            pl.BlockSpec((128, 128), lambda i: (i, 0)),
        ],
        out_specs=[
            pl.BlockSpec((128, 8), lambda i: (i, 0)),
            pl.BlockSpec((128, 8), lambda i: (i, 0)),
            pl.BlockSpec((128, 128), lambda i: (i, 0)),
            pl.BlockSpec((128, 128), lambda i: (i, 0)),
        ],
        out_shape=[jax.ShapeDtypeStruct((NP, 8), f32)] * 2
        + [jax.ShapeDtypeStruct((NP, 128), f32)] * 2,
    )(hs, hd, x_p)


BROW = 1264


def _t1_body(u0, u1, ddd, dds, w1, b1, w2, s2a, s2b):
    z = (u0[0] + u1[0]) * ddd[...][:, 0:1]
    h = jnp.maximum(
        jnp.dot(z, w1[...], preferred_element_type=f32) + b1[...], 0.0)
    s2 = jnp.dot(h, w2[...], preferred_element_type=f32) * dds[...][:, 0:1]
    s2a[...] = s2[:, :128]
    s2b[...] = s2[:, 128:]


def _t1(u, ddd, dds, w1, b1, w2):
    blk = lambda sh: pl.BlockSpec(sh, lambda i: (i, 0))
    full = lambda sh: pl.BlockSpec(sh, lambda i: (0, 0))
    return pl.pallas_call(
        _t1_body,
        grid=(NP // BROW,),
        in_specs=[pl.BlockSpec((1, BROW, 128), lambda i: (0, i, 0)),
                  pl.BlockSpec((1, BROW, 128), lambda i: (1, i, 0)),
                  blk((BROW, 8)), blk((BROW, 8)), full((128, 512)),
                  full((1, 512)), full((512, 256))],
        out_specs=[blk((BROW, 128)), blk((BROW, 128))],
        out_shape=[jax.ShapeDtypeStruct((NP, 128), f32)] * 2,
    )(u, u, ddd, dds, w1, b1, w2)


def _t2_body(u0, u1, ddd, dds, w3, b2, s3_ref, s3b_ref):
    dd = ddd[...][:, 0:1]
    ha = jnp.maximum(u0[0] * dd + b2[...][:, :128], 0.0)
    hb = jnp.maximum(u1[0] * dd + b2[...][:, 128:], 0.0)
    s3 = (jnp.dot(ha, w3[...][:128, :], preferred_element_type=f32)
          + jnp.dot(hb, w3[...][128:, :], preferred_element_type=f32))
    s3v = s3 * dds[...][:, 0:1]
    s3_ref[...] = s3v
    s3b_ref[...] = s3v


def _t2(u, ddd, dds, w3, b2):
    blk = lambda sh: pl.BlockSpec(sh, lambda i: (i, 0))
    full = lambda sh: pl.BlockSpec(sh, lambda i: (0, 0))
    return pl.pallas_call(
        _t2_body,
        grid=(NP // BROW,),
        in_specs=[pl.BlockSpec((1, BROW, 128), lambda i: (0, i, 0)),
                  pl.BlockSpec((1, BROW, 128), lambda i: (1, i, 0)),
                  blk((BROW, 8)), blk((BROW, 8)), full((256, 128)),
                  full((1, 256))],
        out_specs=[blk((BROW, 128)), blk((BROW, 128))],
        out_shape=[jax.ShapeDtypeStruct((NP, 128), f32)] * 2,
    )(u, u, ddd, dds, w3, b2)


def _t3_body(u0, u1, ddd, dds, w4, b3, s4_ref, s4b_ref):
    h = jnp.maximum((u0[0] + u1[0]) * ddd[...][:, 0:1] + b3[...], 0.0)
    s4 = jnp.dot(h, w4[...], preferred_element_type=f32) * dds[...][:, 0:1]
    s4p = jnp.concatenate([s4, jnp.zeros((BROW, 64), f32)], axis=1)
    s4_ref[...] = s4p
    s4b_ref[...] = s4p


def _t3(u, ddd, dds, w4, b3):
    blk = lambda sh: pl.BlockSpec(sh, lambda i: (i, 0))
    full = lambda sh: pl.BlockSpec(sh, lambda i: (0, 0))
    return pl.pallas_call(
        _t3_body,
        grid=(NP // BROW,),
        in_specs=[pl.BlockSpec((1, BROW, 128), lambda i: (0, i, 0)),
                  pl.BlockSpec((1, BROW, 128), lambda i: (1, i, 0)),
                  blk((BROW, 8)), blk((BROW, 8)), full((128, 64)),
                  full((1, 128))],
        out_specs=[blk((BROW, 128)), blk((BROW, 128))],
        out_shape=[jax.ShapeDtypeStruct((NP, 128), f32)] * 2,
    )(u, u, ddd, dds, w4, b3)


def _t4_body(u0, u1, ddd, b4, out_ref):
    u = u0[0][:, :64] + u1[0][:, :64]
    out_ref[...] = u * ddd[...][:, 0:1] + b4[...]


def _t4(u, ddd, b4):
    blk = lambda sh: pl.BlockSpec(sh, lambda i: (i, 0))
    full = lambda sh: pl.BlockSpec(sh, lambda i: (0, 0))
    return pl.pallas_call(
        _t4_body,
        grid=(NP // BROW,),
        in_specs=[pl.BlockSpec((1, BROW, 128), lambda i: (0, i, 0)),
                  pl.BlockSpec((1, BROW, 128), lambda i: (1, i, 0)),
                  blk((BROW, 8)), full((1, 64))],
        out_specs=blk((BROW, 64)),
        out_shape=jax.ShapeDtypeStruct((N, 64), f32),
    )(u, u, ddd, b4)


def kernel(x, edge_index, W1, b1, W2, b2, W3, b3, W4, b4):
    src = edge_index[0].astype(i32)
    dst = edge_index[1].astype(i32)
    pad = N + (jnp.arange(E_PAD - E, dtype=i32) % (NP - N))
    src_f = jnp.concatenate([src, pad])
    dst_f = jnp.concatenate([dst, pad])
    src2 = src_f.reshape(NCHUNKS, CHUNK)
    dst2 = dst_f.reshape(NCHUNKS, CHUNK)
    x_p = jnp.zeros((NP, NFEAT), f32).at[:N, :].set(x)
    z128 = jnp.zeros((ZR, 128), f32)


    hs, hd = _deg(src_f, dst_f)
    dds, ddd, xs, xs2 = _p0(hs, hd, x_p)
    u1 = _agg128(xs, xs2, src2, dst2, z128)
    s2a, s2b = _t1(u1, ddd, dds, W1, b1.reshape(1, -1), W2)
    u2 = _agg128c(s2a, s2b, src2, dst2, z128)
    s3, s3b = _t2(u2, ddd, dds, W3, b2.reshape(1, -1))
    u3 = _agg128(s3, s3b, src2, dst2, z128)
    s4, s4b = _t3(u3, ddd, dds, W4, b3.reshape(1, -1))
    u4 = _agg64(s4, s4b, src2, dst2, z128)
    return _t4(u4, ddd, b4.reshape(1, -1))


# speedup vs baseline: 1.2456x; 1.0437x over previous
import functools

import jax
import jax.numpy as jnp
from jax import lax
from jax.experimental import pallas as pl
from jax.experimental.pallas import tpu as pltpu
from jax.experimental.pallas import tpu_sc as plsc

N = 10000
E = 320000
NFEAT = 128
NP = 10112
CHUNK = 128
NCHUNKS = 2560
E_PAD = NCHUNKS * CHUNK
NLANE = 16
NP2 = 10240
RPT = NP // 16
ZR = 158

f32 = jnp.float32
i32 = jnp.int32


def _mesh():
    return plsc.VectorSubcoreMesh(core_axis_name="c", subcore_axis_name="s")


def _make_deg():
    K = NCHUNKS // 32

    @functools.partial(
        pl.kernel,
        out_type=(jax.ShapeDtypeStruct((32, NP2), f32),
                  jax.ShapeDtypeStruct((32, NP2), f32)),
        mesh=_mesh(),
        scratch_types=[
            pltpu.VMEM((K * CHUNK,), i32),
            pltpu.VMEM((K * CHUNK,), i32),
            pltpu.VMEM((8 * NP,), f32),
            pltpu.VMEM((NP2,), f32),
        ],
        compiler_params=pltpu.CompilerParams(needs_layout_passes=False),
    )
    def deg(src_hbm, dst_hbm, hs_out, hd_out, idx_s, idx_d, hist, osum):
        c = lax.axis_index("c")
        s = lax.axis_index("s")
        wid = s * 2 + c
        base = wid * (K * CHUNK)
        pltpu.sync_copy(src_hbm.at[pl.ds(base, K * CHUNK)], idx_s)
        pltpu.sync_copy(dst_hbm.at[pl.ds(base, K * CHUNK)], idx_d)

        def zstep(i, carry):
            hist[pl.ds(i * NLANE, NLANE)] = jnp.zeros((NLANE,), f32)
            return carry

        lax.fori_loop(0, (8 * NP) // NLANE, zstep, 0)

        for i in range((NP2 - NP) // NLANE):
            osum[pl.ds(NP + i * NLANE, NLANE)] = jnp.zeros((NLANE,), f32)

        iota = lax.iota(i32, NLANE)
        rowoff = (iota % 4) * NP
        ones = jnp.ones((NLANE,), f32)
        masks = [(iota >= q * 4) & (iota < q * 4 + 4) for q in range(4)]

        def estep(j, carry):
            off = j * CHUNK
            for g in range(CHUNK // NLANE):
                sv = idx_s[pl.ds(off + g * NLANE, NLANE)]
                dv = idx_d[pl.ds(off + g * NLANE, NLANE)]
                fs = rowoff + sv
                fd = rowoff + dv + 4 * NP
                for q in range(4):
                    plsc.addupdate_scatter(hist, [fs], ones, mask=masks[q])
                    plsc.addupdate_scatter(hist, [fd], ones, mask=masks[q])
            return carry

        lax.fori_loop(0, K, estep, 0)

        def rstep_src(i, carry):
            off = i * NLANE
            a = (hist[pl.ds(off, NLANE)] + hist[pl.ds(NP + off, NLANE)]
                 + hist[pl.ds(2 * NP + off, NLANE)]
                 + hist[pl.ds(3 * NP + off, NLANE)])
            osum[pl.ds(off, NLANE)] = a
            return carry

        lax.fori_loop(0, NP // NLANE, rstep_src, 0)
        pltpu.sync_copy(osum, hs_out.at[wid])

        def rstep_dst(i, carry):
            off = i * NLANE
            a = (hist[pl.ds(4 * NP + off, NLANE)]
                 + hist[pl.ds(5 * NP + off, NLANE)]
                 + hist[pl.ds(6 * NP + off, NLANE)]
                 + hist[pl.ds(7 * NP + off, NLANE)])
            osum[pl.ds(off, NLANE)] = a
            return carry

        lax.fori_loop(0, NP // NLANE, rstep_dst, 0)
        pltpu.sync_copy(osum, hd_out.at[wid])

    return deg


IB = 8


def _make_agg(D, col_split):
    K = NCHUNKS // 16 if col_split else NCHUNKS // 32

    @functools.partial(
        pl.kernel,
        out_type=jax.ShapeDtypeStruct((2, NP, D), f32),
        mesh=_mesh(),
        scratch_types=[
            pltpu.VMEM((IB, CHUNK), i32),
            pltpu.VMEM((IB, CHUNK), i32),
            pltpu.VMEM((CHUNK, D), f32),
            pltpu.VMEM((CHUNK, D), f32),
            pltpu.VMEM_SHARED((NP, D), f32),
            pltpu.SemaphoreType.DMA,
            pltpu.SemaphoreType.DMA,
            pltpu.SemaphoreType.DMA,
            pltpu.SemaphoreType.DMA,
        ],
    )
    def agg(tbl0, tbl1, src_hbm, dst_hbm, zeros_hbm, out,
            idx_s, idx_d, rows0, rows1, aggm, gsem0, gsem1, ssem0, ssem1):
        c = lax.axis_index("c")
        s = lax.axis_index("s")
        if col_split:
            base = s * K
        else:
            base = (s * 2 + c) * K
        for q in range(RPT // ZR):
            pltpu.sync_copy(zeros_hbm, aggm.at[pl.ds(s * RPT + q * ZR, ZR)])
        plsc.subcore_barrier()

        rows = (rows0, rows1)
        gsem = (gsem0, gsem1)
        ssem = (ssem0, ssem1)

        def run(tbl):
            def outer(t, carry):
                pltpu.sync_copy(src_hbm.at[pl.ds(base + t * IB, IB)], idx_s)
                pltpu.sync_copy(dst_hbm.at[pl.ds(base + t * IB, IB)], idx_d)
                g = [None] * IB
                sd = [None] * IB
                g[0] = pltpu.async_copy(tbl.at[idx_s.at[0]], rows[0], gsem[0])
                for j in range(IB):
                    if j >= 1:
                        sd[j - 1].wait()
                    if j + 1 < IB:
                        g[j + 1] = pltpu.async_copy(
                            tbl.at[idx_s.at[j + 1]], rows[(j + 1) % 2],
                            gsem[(j + 1) % 2])
                    g[j].wait()
                    sd[j] = pltpu.async_copy(
                        rows[j % 2], aggm.at[idx_d.at[j]], ssem[j % 2],
                        add=True)
                sd[IB - 1].wait()
                return carry
            lax.fori_loop(0, K // IB, outer, 0)

        @pl.when(c == 0)
        def _():
            run(tbl0)

        @pl.when(c == 1)
        def _():
            run(tbl1)

        plsc.subcore_barrier()
        pltpu.sync_copy(aggm.at[pl.ds(s * RPT, RPT)],
                        out.at[c, pl.ds(s * RPT, RPT)])

    return agg


_deg = _make_deg()
_agg128 = _make_agg(128, False)
_agg128c = _make_agg(128, True)
_agg64 = _make_agg(128, False)


PB = 1280


def _p0_body(hs_ref, hd_ref, x_ref, dds_ref, ddd_ref, xs_ref, xs2_ref):
    rs = lax.rsqrt(jnp.maximum(jnp.sum(hs_ref[...], axis=0), 1.0))
    rd = lax.rsqrt(jnp.maximum(jnp.sum(hd_ref[...], axis=0), 1.0))
    dsb = jnp.transpose(jnp.broadcast_to(rs[None, :], (128, PB)))
    ddb = jnp.transpose(jnp.broadcast_to(rd[None, :], (128, PB)))
    dds_ref[...] = dsb[:, :8]
    ddd_ref[...] = ddb[:, :8]
    xs = x_ref[...] * dsb
    xs_ref[...] = xs
    xs2_ref[...] = xs


def _p0(hs, hd, x_p):
    return pl.pallas_call(
        _p0_body,
        grid=(NP2 // PB,),
        in_specs=[
            pl.BlockSpec((32, PB), lambda i: (0, i)),
            pl.BlockSpec((32, PB), lambda i: (0, i)),
            pl.BlockSpec((PB, 128), lambda i: (i, 0)),
        ],
        out_specs=[
            pl.BlockSpec((PB, 8), lambda i: (i, 0)),
            pl.BlockSpec((PB, 8), lambda i: (i, 0)),
            pl.BlockSpec((PB, 128), lambda i: (i, 0)),
            pl.BlockSpec((PB, 128), lambda i: (i, 0)),
        ],
        out_shape=[jax.ShapeDtypeStruct((NP, 8), f32)] * 2
        + [jax.ShapeDtypeStruct((NP, 128), f32)] * 2,
    )(hs, hd, x_p)


BROW = 1264


def _t1_body(u0, u1, ddd, dds, w1, b1, w2, s2a, s2b):
    z = (u0[0] + u1[0]) * ddd[...][:, 0:1]
    h = jnp.maximum(
        jnp.dot(z, w1[...], preferred_element_type=f32) + b1[...], 0.0)
    s2 = jnp.dot(h, w2[...], preferred_element_type=f32) * dds[...][:, 0:1]
    s2a[...] = s2[:, :128]
    s2b[...] = s2[:, 128:]


def _t1(u, ddd, dds, w1, b1, w2):
    blk = lambda sh: pl.BlockSpec(sh, lambda i: (i, 0))
    full = lambda sh: pl.BlockSpec(sh, lambda i: (0, 0))
    return pl.pallas_call(
        _t1_body,
        grid=(NP // BROW,),
        in_specs=[pl.BlockSpec((1, BROW, 128), lambda i: (0, i, 0)),
                  pl.BlockSpec((1, BROW, 128), lambda i: (1, i, 0)),
                  blk((BROW, 8)), blk((BROW, 8)), full((128, 512)),
                  full((1, 512)), full((512, 256))],
        out_specs=[blk((BROW, 128)), blk((BROW, 128))],
        out_shape=[jax.ShapeDtypeStruct((NP, 128), f32)] * 2,
    )(u, u, ddd, dds, w1, b1, w2)


def _t2_body(u0, u1, ddd, dds, w3, b2, s3_ref, s3b_ref):
    dd = ddd[...][:, 0:1]
    ha = jnp.maximum(u0[0] * dd + b2[...][:, :128], 0.0)
    hb = jnp.maximum(u1[0] * dd + b2[...][:, 128:], 0.0)
    s3 = (jnp.dot(ha, w3[...][:128, :], preferred_element_type=f32)
          + jnp.dot(hb, w3[...][128:, :], preferred_element_type=f32))
    s3v = s3 * dds[...][:, 0:1]
    s3_ref[...] = s3v
    s3b_ref[...] = s3v


def _t2(u, ddd, dds, w3, b2):
    blk = lambda sh: pl.BlockSpec(sh, lambda i: (i, 0))
    full = lambda sh: pl.BlockSpec(sh, lambda i: (0, 0))
    return pl.pallas_call(
        _t2_body,
        grid=(NP // BROW,),
        in_specs=[pl.BlockSpec((1, BROW, 128), lambda i: (0, i, 0)),
                  pl.BlockSpec((1, BROW, 128), lambda i: (1, i, 0)),
                  blk((BROW, 8)), blk((BROW, 8)), full((256, 128)),
                  full((1, 256))],
        out_specs=[blk((BROW, 128)), blk((BROW, 128))],
        out_shape=[jax.ShapeDtypeStruct((NP, 128), f32)] * 2,
    )(u, u, ddd, dds, w3, b2)


def _t3_body(u0, u1, ddd, dds, w4, b3, s4_ref, s4b_ref):
    h = jnp.maximum((u0[0] + u1[0]) * ddd[...][:, 0:1] + b3[...], 0.0)
    s4 = jnp.dot(h, w4[...], preferred_element_type=f32) * dds[...][:, 0:1]
    s4p = jnp.concatenate([s4, jnp.zeros((BROW, 64), f32)], axis=1)
    s4_ref[...] = s4p
    s4b_ref[...] = s4p


def _t3(u, ddd, dds, w4, b3):
    blk = lambda sh: pl.BlockSpec(sh, lambda i: (i, 0))
    full = lambda sh: pl.BlockSpec(sh, lambda i: (0, 0))
    return pl.pallas_call(
        _t3_body,
        grid=(NP // BROW,),
        in_specs=[pl.BlockSpec((1, BROW, 128), lambda i: (0, i, 0)),
                  pl.BlockSpec((1, BROW, 128), lambda i: (1, i, 0)),
                  blk((BROW, 8)), blk((BROW, 8)), full((128, 64)),
                  full((1, 128))],
        out_specs=[blk((BROW, 128)), blk((BROW, 128))],
        out_shape=[jax.ShapeDtypeStruct((NP, 128), f32)] * 2,
    )(u, u, ddd, dds, w4, b3)


def _t4_body(u0, u1, ddd, b4, out_ref):
    u = u0[0][:, :64] + u1[0][:, :64]
    out_ref[...] = u * ddd[...][:, 0:1] + b4[...]


def _t4(u, ddd, b4):
    blk = lambda sh: pl.BlockSpec(sh, lambda i: (i, 0))
    full = lambda sh: pl.BlockSpec(sh, lambda i: (0, 0))
    return pl.pallas_call(
        _t4_body,
        grid=(NP // BROW,),
        in_specs=[pl.BlockSpec((1, BROW, 128), lambda i: (0, i, 0)),
                  pl.BlockSpec((1, BROW, 128), lambda i: (1, i, 0)),
                  blk((BROW, 8)), full((1, 64))],
        out_specs=blk((BROW, 64)),
        out_shape=jax.ShapeDtypeStruct((N, 64), f32),
    )(u, u, ddd, b4)


def kernel(x, edge_index, W1, b1, W2, b2, W3, b3, W4, b4):
    src = edge_index[0].astype(i32)
    dst = edge_index[1].astype(i32)
    pad = N + (jnp.arange(E_PAD - E, dtype=i32) % (NP - N))
    src_f = jnp.concatenate([src, pad])
    dst_f = jnp.concatenate([dst, pad])
    src2 = src_f.reshape(NCHUNKS, CHUNK)
    dst2 = dst_f.reshape(NCHUNKS, CHUNK)
    x_p = jnp.zeros((NP, NFEAT), f32).at[:N, :].set(x)
    z128 = jnp.zeros((ZR, 128), f32)


    hs, hd = _deg(src_f, dst_f)
    dds, ddd, xs, xs2 = _p0(hs, hd, x_p)
    u1 = _agg128(xs, xs2, src2, dst2, z128)
    s2a, s2b = _t1(u1, ddd, dds, W1, b1.reshape(1, -1), W2)
    u2 = _agg128c(s2a, s2b, src2, dst2, z128)
    s3, s3b = _t2(u2, ddd, dds, W3, b2.reshape(1, -1))
    u3 = _agg128(s3, s3b, src2, dst2, z128)
    s4, s4b = _t3(u3, ddd, dds, W4, b3.reshape(1, -1))
    u4 = _agg64(s4, s4b, src2, dst2, z128)
    return _t4(u4, ddd, b4.reshape(1, -1))
